# baseline (speedup 1.0000x reference)
import jax
import jax.numpy as jnp
from jax import lax
from jax.experimental import pallas as pl
from jax.experimental.pallas import tpu as pltpu
from jax.experimental.pallas import tpu_sc as plsc

_B = 16384
_ND = 13
_NS = 26
_VOCAB = 100000
_XC = _ND + _NS
_NW = 32
_RPW = _B // _NW
_NCHUNK = _NS * _RPW // 128
_SPLITS = (18, 8)


def _wid(c, s):
    return s * 2 + c


def _body_a(xt_hbm, w_hbm, idx_hbm, dense_hbm, xcv, idxv, wv, accv, sem):
    wid = _wid(lax.axis_index("c"), lax.axis_index("s"))
    base = wid * _RPW

    def stage_body(j, carry):
        pltpu.make_async_copy(
            xt_hbm.at[pl.ds(j * _B + base, _RPW)], xcv.at[j], sem
        ).start()
        return carry

    lax.fori_loop(0, _XC, stage_body, 0)
    pltpu.sync_copy(w_hbm, wv)

    def stage_wait(j, carry):
        pltpu.make_async_copy(
            xt_hbm.at[pl.ds(j * _B + base, _RPW)], xcv.at[j], sem
        ).wait()
        return carry

    lax.fori_loop(0, _XC, stage_wait, 0)

    def idx_body(cc, carry):
        f = cc // 4
        r0 = (cc % 4) * 128
        for vv in range(8):
            vals = xcv[_ND + f, pl.ds(r0 + vv * 16, 16)]
            idxv[cc, pl.ds(vv * 16, 16)] = vals.astype(jnp.int32)
        return carry

    lax.fori_loop(0, _NCHUNK, idx_body, 0)

    wvec = wv[pl.ds(0, 16)]

    def dense_body(cc, carry):
        for vv in range(8):
            b0 = cc * 128 + vv * 16
            acc = xcv[0, pl.ds(b0, 16)] * wvec[0]
            for j in range(1, _ND):
                acc = acc + xcv[j, pl.ds(b0, 16)] * wvec[j]
            accv[pl.ds(b0, 16)] = acc
        return carry

    lax.fori_loop(0, 4, dense_body, 0)

    pltpu.sync_copy(idxv, idx_hbm.at[pl.ds(wid * _NCHUNK, _NCHUNK)])
    pltpu.sync_copy(accv, dense_hbm.at[pl.ds(base, _RPW)])


def _make_body_b(f0, nf):
    nch = nf * 4

    def body(idx_hbm, *rest):
        tbls = rest[:nf]
        part_hbm, out_hbm, idxv, gv, accv, sem = rest[nf:]
        wid = _wid(lax.axis_index("c"), lax.axis_index("s"))
        base = wid * _RPW

        pltpu.sync_copy(
            idx_hbm.at[pl.ds(wid * _NCHUNK + f0 * 4, nch)], idxv
        )
        pltpu.make_async_copy(
            part_hbm.at[pl.ds(base, _RPW)], accv, sem
        ).start()

        for k in range(nf):
            def fire_body(cc, carry, k=k):
                pltpu.make_async_copy(
                    tbls[k].at[idxv.at[k * 4 + cc]], gv.at[k * 4 + cc], sem
                ).start()
                return carry

            lax.fori_loop(0, 4, fire_body, 0)

        pltpu.make_async_copy(
            part_hbm.at[pl.ds(base, _RPW)], accv, sem
        ).wait()

        def drain_body(cc, carry):
            pltpu.make_async_copy(
                tbls[0].at[idxv.at[cc]], gv.at[cc], sem
            ).wait()
            return carry

        lax.fori_loop(0, nch, drain_body, 0)

        def acc_body(cc, carry):
            for vv in range(8):
                b0 = cc * 128 + vv * 16
                acc = accv[pl.ds(b0, 16)]
                for k in range(nf):
                    acc = acc + gv[k * 4 + cc, pl.ds(vv * 16, 16)]
                accv[pl.ds(b0, 16)] = acc
            return carry

        lax.fori_loop(0, 4, acc_body, 0)

        pltpu.sync_copy(accv, out_hbm.at[pl.ds(base, _RPW)])

    return body


@jax.jit
def _run(xt_flat, tbl_list, w_pad):
    mesh = plsc.VectorSubcoreMesh(
        core_axis_name="c", subcore_axis_name="s", num_cores=2, num_subcores=16
    )
    phase_a = pl.kernel(
        _body_a,
        out_type=(
            jax.ShapeDtypeStruct((_NW * _NCHUNK, 128), jnp.int32),
            jax.ShapeDtypeStruct((_B,), jnp.float32),
        ),
        mesh=mesh,
        scratch_types=[
            pltpu.VMEM((_XC, _RPW), jnp.float32),
            pltpu.VMEM((_NCHUNK, 128), jnp.int32),
            pltpu.VMEM((16,), jnp.float32),
            pltpu.VMEM((_RPW,), jnp.float32),
            pltpu.SemaphoreType.DMA,
        ],
    )

    def make_phase_b(f0, nf):
        return pl.kernel(
            _make_body_b(f0, nf),
            out_type=jax.ShapeDtypeStruct((_B,), jnp.float32),
            mesh=mesh,
            scratch_types=[
                pltpu.VMEM((nf * 4, 128), jnp.int32),
                pltpu.VMEM((nf * 4, 128), jnp.float32),
                pltpu.VMEM((_RPW,), jnp.float32),
                pltpu.SemaphoreType.DMA,
            ],
        )

    idx_hbm, part = phase_a(xt_flat, w_pad)
    f0 = 0
    for nf in _SPLITS:
        part = make_phase_b(f0, nf)(idx_hbm, *tbl_list[f0:f0 + nf], part)
        f0 += nf
    return part


def kernel(X, tables, weight):
    xt_flat = X.T.reshape(_XC * _B)
    tbl_list = []
    src = tables
    f0 = 0
    for nf in _SPLITS:
        tbl_list += [src[f, :, 0] for f in range(f0, f0 + nf)]
        f0 += nf
        src = lax.optimization_barrier(src)
    w_pad = jnp.pad(weight.reshape(_ND), (0, 16 - _ND))
    return _run(xt_flat, tbl_list, w_pad).reshape(_B, 1)


# speedup vs baseline: 1.0514x; 1.0514x over previous
import jax
import jax.numpy as jnp
from jax import lax
from jax.experimental import pallas as pl
from jax.experimental.pallas import tpu as pltpu
from jax.experimental.pallas import tpu_sc as plsc

_B = 16384
_ND = 13
_NS = 26
_VOCAB = 100000
_XC = _ND + _NS
_NW = 32
_RPW = _B // _NW
_NCHUNK = _NS * _RPW // 128
_SPLITS = (12, 8, 6)


def _wid(c, s):
    return s * 2 + c


def _body_a(xt_hbm, w_hbm, idx_hbm, dense_hbm, xcv, idxv, wv, accv, sem):
    wid = _wid(lax.axis_index("c"), lax.axis_index("s"))
    base = wid * _RPW

    def stage_body(j, carry):
        pltpu.make_async_copy(
            xt_hbm.at[pl.ds(j * _B + base, _RPW)], xcv.at[j], sem
        ).start()
        return carry

    lax.fori_loop(0, _XC, stage_body, 0)
    pltpu.sync_copy(w_hbm, wv)

    def stage_wait(j, carry):
        pltpu.make_async_copy(
            xt_hbm.at[pl.ds(j * _B + base, _RPW)], xcv.at[j], sem
        ).wait()
        return carry

    lax.fori_loop(0, _XC, stage_wait, 0)

    def idx_body(cc, carry):
        f = cc // 4
        r0 = (cc % 4) * 128
        for vv in range(8):
            vals = xcv[_ND + f, pl.ds(r0 + vv * 16, 16)]
            idxv[cc, pl.ds(vv * 16, 16)] = vals.astype(jnp.int32)
        return carry

    lax.fori_loop(0, _NCHUNK, idx_body, 0)

    wvec = wv[pl.ds(0, 16)]

    def dense_body(cc, carry):
        for vv in range(8):
            b0 = cc * 128 + vv * 16
            acc = xcv[0, pl.ds(b0, 16)] * wvec[0]
            for j in range(1, _ND):
                acc = acc + xcv[j, pl.ds(b0, 16)] * wvec[j]
            accv[pl.ds(b0, 16)] = acc
        return carry

    lax.fori_loop(0, 4, dense_body, 0)

    pltpu.sync_copy(idxv, idx_hbm.at[pl.ds(wid * _NCHUNK, _NCHUNK)])
    pltpu.sync_copy(accv, dense_hbm.at[pl.ds(base, _RPW)])


def _make_body_b(f0, nf):
    nch = nf * 4

    def body(idx_hbm, *rest):
        tbls = rest[:nf]
        part_hbm, out_hbm, idxv, gv, accv, sem = rest[nf:]
        wid = _wid(lax.axis_index("c"), lax.axis_index("s"))
        base = wid * _RPW

        pltpu.sync_copy(
            idx_hbm.at[pl.ds(wid * _NCHUNK + f0 * 4, nch)], idxv
        )
        pltpu.make_async_copy(
            part_hbm.at[pl.ds(base, _RPW)], accv, sem
        ).start()

        for k in range(nf):
            def fire_body(cc, carry, k=k):
                pltpu.make_async_copy(
                    tbls[k].at[idxv.at[k * 4 + cc]], gv.at[k * 4 + cc], sem
                ).start()
                return carry

            lax.fori_loop(0, 4, fire_body, 0)

        pltpu.make_async_copy(
            part_hbm.at[pl.ds(base, _RPW)], accv, sem
        ).wait()

        def drain_body(cc, carry):
            pltpu.make_async_copy(
                tbls[0].at[idxv.at[cc]], gv.at[cc], sem
            ).wait()
            return carry

        lax.fori_loop(0, nch, drain_body, 0)

        def acc_body(cc, carry):
            for vv in range(8):
                b0 = cc * 128 + vv * 16
                acc = accv[pl.ds(b0, 16)]
                for k in range(nf):
                    acc = acc + gv[k * 4 + cc, pl.ds(vv * 16, 16)]
                accv[pl.ds(b0, 16)] = acc
            return carry

        lax.fori_loop(0, 4, acc_body, 0)

        pltpu.sync_copy(accv, out_hbm.at[pl.ds(base, _RPW)])

    return body


@jax.jit
def _run(xt_flat, tbl_list, w_pad):
    mesh = plsc.VectorSubcoreMesh(
        core_axis_name="c", subcore_axis_name="s", num_cores=2, num_subcores=16
    )
    phase_a = pl.kernel(
        _body_a,
        out_type=(
            jax.ShapeDtypeStruct((_NW * _NCHUNK, 128), jnp.int32),
            jax.ShapeDtypeStruct((_B,), jnp.float32),
        ),
        mesh=mesh,
        scratch_types=[
            pltpu.VMEM((_XC, _RPW), jnp.float32),
            pltpu.VMEM((_NCHUNK, 128), jnp.int32),
            pltpu.VMEM((16,), jnp.float32),
            pltpu.VMEM((_RPW,), jnp.float32),
            pltpu.SemaphoreType.DMA,
        ],
    )

    def make_phase_b(f0, nf):
        return pl.kernel(
            _make_body_b(f0, nf),
            out_type=jax.ShapeDtypeStruct((_B,), jnp.float32),
            mesh=mesh,
            scratch_types=[
                pltpu.VMEM((nf * 4, 128), jnp.int32),
                pltpu.VMEM((nf * 4, 128), jnp.float32),
                pltpu.VMEM((_RPW,), jnp.float32),
                pltpu.SemaphoreType.DMA,
            ],
        )

    idx_hbm, part = phase_a(xt_flat, w_pad)
    f0 = 0
    for nf in _SPLITS:
        part = make_phase_b(f0, nf)(idx_hbm, *tbl_list[f0:f0 + nf], part)
        f0 += nf
    return part


def kernel(X, tables, weight):
    xt_flat = X.T.reshape(_XC * _B)
    tbl_list = []
    src = tables
    f0 = 0
    for nf in _SPLITS:
        tbl_list += [src[f, :, 0] for f in range(f0, f0 + nf)]
        f0 += nf
        src = lax.optimization_barrier(src)
    w_pad = jnp.pad(weight.reshape(_ND), (0, 16 - _ND))
    return _run(xt_flat, tbl_list, w_pad).reshape(_B, 1)
